# speedup vs baseline: 4.3389x; 4.3389x over previous
import functools

import jax
import jax.numpy as jnp
from jax import lax
from jax.experimental import pallas as pl
from jax.experimental.pallas import tpu as pltpu
from jax.experimental.pallas import tpu_sc as plsc

N_NODES = 100000
D = 128
S = 512
TILE = 128
NUM_TILES = N_NODES // TILE
TAIL = N_NODES - NUM_TILES * TILE
NC = 2
NS = 16
NW = NC * NS
ROWS_PER_SUBCORE = S // NS


def _sc_segment_partials(x, batch):
    mesh = plsc.VectorSubcoreMesh(core_axis_name="c", subcore_axis_name="s")

    @functools.partial(
        pl.kernel,
        out_type=jax.ShapeDtypeStruct((NC, S, D), jnp.float32),
        mesh=mesh,
        scratch_types=[
            pltpu.VMEM((TILE,), jnp.int32),
            pltpu.VMEM((TILE, D), jnp.float32),
            pltpu.VMEM((ROWS_PER_SUBCORE, D), jnp.float32),
            pltpu.VMEM_SHARED((S, D), jnp.float32),
        ],
    )
    def k(x_hbm, b_hbm, out_hbm, idx_v, rows_v, zb_v, acc_sh):
        c = lax.axis_index("c")
        s = lax.axis_index("s")
        wid = s * NC + c

        @pl.loop(0, ROWS_PER_SUBCORE)
        def _(r):
            for v in range(D // 16):
                zb_v[r, pl.ds(v * 16, 16)] = jnp.zeros((16,), jnp.float32)

        pltpu.sync_copy(zb_v, acc_sh.at[pl.ds(s * ROWS_PER_SUBCORE, ROWS_PER_SUBCORE)])
        plsc.subcore_barrier()

        @pl.loop(wid, NUM_TILES, step=NW)
        def _(t):
            base = t * TILE
            pltpu.sync_copy(b_hbm.at[pl.ds(base, TILE)], idx_v)
            pltpu.sync_copy(x_hbm.at[pl.ds(base, TILE)], rows_v)
            pltpu.sync_copy(rows_v, acc_sh.at[idx_v], add=True)

        plsc.subcore_barrier()

        sl = pl.ds(s * ROWS_PER_SUBCORE, ROWS_PER_SUBCORE)
        pltpu.sync_copy(acc_sh.at[sl], out_hbm.at[c, sl])

    return k(x, batch)


def _tc_finish(parts, tail_x, tail_ids, W, b):
    def body(p_ref, tx_ref, ti_ref, w_ref, b_ref, o_ref):
        ids = ti_ref[...]
        iota = lax.broadcasted_iota(jnp.int32, (S, TAIL), 0)
        onehot = (iota == ids).astype(jnp.float32)
        pooled = p_ref[0] + p_ref[1]
        pooled = pooled + lax.dot_general(
            onehot, tx_ref[...], (((1,), (0,)), ((), ())),
            preferred_element_type=jnp.float32)
        o_ref[...] = lax.dot_general(
            pooled, w_ref[...], (((1,), (1,)), ((), ())),
            preferred_element_type=jnp.float32) + b_ref[...]

    return pl.pallas_call(
        body,
        out_shape=jax.ShapeDtypeStruct((S, D), jnp.float32),
    )(parts, tail_x, tail_ids, W, b)


def kernel(x, batch, W, b):
    batch = batch.astype(jnp.int32)
    parts = _sc_segment_partials(x, batch)
    tail_x = x[NUM_TILES * TILE:]
    tail_ids = batch[NUM_TILES * TILE:].reshape(1, TAIL)
    return _tc_finish(parts, tail_x, tail_ids, W, b.reshape(1, D))


# speedup vs baseline: 6.7215x; 1.5491x over previous
import functools

import jax
import jax.numpy as jnp
from jax import lax
from jax.experimental import pallas as pl
from jax.experimental.pallas import tpu as pltpu
from jax.experimental.pallas import tpu_sc as plsc

N_NODES = 100000
D = 128
S = 512
TILE = 128
NUM_TILES = N_NODES // TILE
TAIL = N_NODES - NUM_TILES * TILE
NC = 2
NS = 16
NW = NC * NS
ROWS_PER_SUBCORE = S // NS


def _sc_segment_partials(x, batch):
    mesh = plsc.VectorSubcoreMesh(core_axis_name="c", subcore_axis_name="s")

    base_tiles = NUM_TILES // NW
    rem_tiles = NUM_TILES - base_tiles * NW
    max_tiles = base_tiles + 1
    NBUF = 4

    @functools.partial(
        pl.kernel,
        out_type=jax.ShapeDtypeStruct((NC, S, D), jnp.float32),
        mesh=mesh,
        scratch_types=[
            pltpu.VMEM((max_tiles, 1, TILE), jnp.int32),
            pltpu.VMEM((NBUF, TILE, D), jnp.float32),
            pltpu.VMEM((ROWS_PER_SUBCORE, D), jnp.float32),
            pltpu.VMEM_SHARED((S, D), jnp.float32),
            pltpu.SemaphoreType.DMA((NBUF,)),
            pltpu.SemaphoreType.DMA((NBUF,)),
        ],
    )
    def k(x_hbm, b_hbm, out_hbm, idx_v, rows_v, zb_v, acc_sh, lsems, ssems):
        c = lax.axis_index("c")
        s = lax.axis_index("s")
        wid = s * NC + c
        start = wid * base_tiles + jnp.minimum(wid, rem_tiles)
        cnt = jnp.where(wid < rem_tiles, base_tiles + 1, base_tiles)

        def issue_load(j):
            pltpu.async_copy(x_hbm.at[pl.ds((start + j) * TILE, TILE)],
                             rows_v.at[j % NBUF], lsems.at[j % NBUF])

        def wait_equal_tile(sem):
            pltpu.make_async_copy(x_hbm.at[pl.ds(0, TILE)], rows_v.at[0],
                                  sem).wait()

        pltpu.sync_copy(b_hbm.at[pl.ds(start, base_tiles)],
                        idx_v.at[pl.ds(0, base_tiles)])

        @pl.when(wid < rem_tiles)
        def _():
            pltpu.sync_copy(b_hbm.at[pl.ds(start + base_tiles, 1)],
                            idx_v.at[pl.ds(base_tiles, 1)])

        for j in range(min(NBUF, max_tiles)):
            @pl.when(j < cnt)
            def _(j=j):
                issue_load(j)

        @pl.loop(0, ROWS_PER_SUBCORE)
        def _(r):
            for v in range(D // 16):
                zb_v[r, pl.ds(v * 16, 16)] = jnp.zeros((16,), jnp.float32)

        pltpu.sync_copy(zb_v, acc_sh.at[pl.ds(s * ROWS_PER_SUBCORE, ROWS_PER_SUBCORE)])
        plsc.subcore_barrier()

        for j in range(max_tiles):
            b = j % NBUF

            @pl.when(j < cnt)
            def _(j=j, b=b):
                wait_equal_tile(lsems.at[b])
                pltpu.async_copy(rows_v.at[b], acc_sh.at[idx_v.at[j, 0]],
                                 ssems.at[b], add=True)

            if j + NBUF < max_tiles:
                @pl.when(j + NBUF < cnt)
                def _(j=j, b=b):
                    wait_equal_tile(ssems.at[b])
                    issue_load(j + NBUF)

        for j in range(max_tiles):
            @pl.when((j >= cnt - NBUF) & (j < cnt))
            def _(j=j):
                wait_equal_tile(ssems.at[j % NBUF])

        plsc.subcore_barrier()

        sl = pl.ds(s * ROWS_PER_SUBCORE, ROWS_PER_SUBCORE)
        pltpu.sync_copy(acc_sh.at[sl], out_hbm.at[c, sl])

    return k(x, batch)


def _tc_finish(parts, tail_x, tail_ids, W, b):
    def body(p_ref, tx_ref, ti_ref, w_ref, b_ref, o_ref):
        ids = ti_ref[...]
        iota = lax.broadcasted_iota(jnp.int32, (S, TAIL), 0)
        onehot = (iota == ids).astype(jnp.float32)
        pooled = p_ref[0] + p_ref[1]
        pooled = pooled + lax.dot_general(
            onehot, tx_ref[...], (((1,), (0,)), ((), ())),
            preferred_element_type=jnp.float32)
        o_ref[...] = lax.dot_general(
            pooled, w_ref[...], (((1,), (1,)), ((), ())),
            preferred_element_type=jnp.float32) + b_ref[...]

    return pl.pallas_call(
        body,
        out_shape=jax.ShapeDtypeStruct((S, D), jnp.float32),
    )(parts, tail_x, tail_ids, W, b)


def kernel(x, batch, W, b):
    batch = batch.astype(jnp.int32)
    batch_tiles = batch[:NUM_TILES * TILE].reshape(NUM_TILES, 1, TILE)
    parts = _sc_segment_partials(x, batch_tiles)
    tail_x = x[NUM_TILES * TILE:]
    tail_ids = batch[NUM_TILES * TILE:].reshape(1, TAIL)
    return _tc_finish(parts, tail_x, tail_ids, W, b.reshape(1, D))
